# baseline (speedup 1.0000x reference)
import functools

import jax
import jax.numpy as jnp
from jax import lax
from jax.experimental import pallas as pl
from jax.experimental.pallas import tpu as pltpu
from jax.experimental.pallas import tpu_sc as plsc

_S = 30.0
_LOG2E = 1.4426950408889634
_LN2 = 0.6931471805599453
_K = _S * _LOG2E
_B = 16384
_C = 100
_L = 16
_NC = 2
_NS = 16
_NW = _NC * _NS
_RW = _B // _NW
_G = _RW // _L


def _ln(s):
    bits = plsc.bitcast(s, jnp.int32)
    e = (bits >> 23) - 127
    mant = plsc.bitcast((bits & 0x7FFFFF) | 0x3F800000, jnp.float32)
    u = mant - 1.0
    y = e.astype(jnp.float32) * 0.6931472 + u * (1.0 - u * (0.5 - u * 0.33333334))
    for _ in range(3):
        y = y + s * jnp.exp(-y) - 1.0
    return y


def _sc_body(x_hbm, m_hbm, t_hbm, out_hbm, x_v, t_v, m_v, mx_v, o_v):
    wid = lax.axis_index("s") * _NC + lax.axis_index("c")
    base = wid * _RW
    pltpu.sync_copy(x_hbm.at[pl.ds(base, _RW), :], x_v)
    pltpu.sync_copy(t_hbm.at[pl.ds(base, _RW)], t_v)
    pltpu.sync_copy(m_hbm, m_v)
    lanes = lax.iota(jnp.int32, _L)

    @plsc.parallel_loop(0, _G, unroll=2)
    def pass1(g):
        r0 = g * _L
        rows = lanes + r0

        def col(c):
            return plsc.load_gather(x_v, [rows, jnp.full((_L,), c, jnp.int32)])

        m0 = m1 = m2 = m3 = jnp.full((_L,), -3.0e38, jnp.float32)
        for c in range(0, _C, 4):
            a = col(c)
            b = col(c + 1)
            d = col(c + 2)
            e = col(c + 3)
            m0 = jnp.maximum(m0, a)
            m1 = jnp.maximum(m1, b)
            m2 = jnp.maximum(m2, d)
            m3 = jnp.maximum(m3, e)
        mx = jnp.maximum(jnp.maximum(m0, m1), jnp.maximum(m2, m3))
        mx_v[pl.ds(r0, _L)] = mx

    @plsc.parallel_loop(0, _G, unroll=2)
    def pass2(g):
        r0 = g * _L
        rows = lanes + r0
        t = t_v[pl.ds(r0, _L)]
        bm = plsc.load_gather(m_v, [t])
        mx = mx_v[pl.ds(r0, _L)]

        def col(c):
            return plsc.load_gather(x_v, [rows, jnp.full((_L,), c, jnp.int32)])

        s0 = s1 = s2 = s3 = jnp.zeros((_L,), jnp.float32)
        for c in range(0, _C, 4):
            a = col(c)
            b = col(c + 1)
            d = col(c + 2)
            e = col(c + 3)
            s0 = s0 + jnp.exp((a - mx) * _S)
            s1 = s1 + jnp.exp((b - mx) * _S)
            s2 = s2 + jnp.exp((d - mx) * _S)
            s3 = s3 + jnp.exp((e - mx) * _S)
        s = (s0 + s1) + (s2 + s3)

        xt = plsc.load_gather(x_v, [rows, t])
        e_raw = jnp.exp((xt - mx) * _S)
        e_mod = jnp.exp((xt - bm - mx) * _S)
        s = jnp.maximum(s - e_raw + e_mod, 1e-30)

        loss = _ln(s) + _S * ((mx - xt) + bm)
        o_v[pl.ds(r0, _L)] = loss

    pltpu.sync_copy(o_v, out_hbm.at[pl.ds(base, _RW)])


_sc_ldam = functools.partial(
    pl.kernel,
    out_type=jax.ShapeDtypeStruct((_B,), jnp.float32),
    mesh=plsc.VectorSubcoreMesh(
        core_axis_name="c", subcore_axis_name="s", num_cores=_NC, num_subcores=_NS
    ),
    scratch_types=[
        pltpu.VMEM((_RW, _C), jnp.float32),
        pltpu.VMEM((_RW,), jnp.int32),
        pltpu.VMEM((_C,), jnp.float32),
        pltpu.VMEM((_RW,), jnp.float32),
        pltpu.VMEM((_RW,), jnp.float32),
    ],
    compiler_params=pltpu.CompilerParams(needs_layout_passes=False),
)(_sc_body)


def _mean_body(x_ref, out_ref):
    out_ref[0, 0] = jnp.sum(x_ref[...]) * (1.0 / _B)


def kernel(inputs, m_list, targets):
    per_row = _sc_ldam(inputs, m_list, targets)
    out = pl.pallas_call(
        _mean_body,
        out_specs=pl.BlockSpec(memory_space=pltpu.SMEM),
        out_shape=jax.ShapeDtypeStruct((1, 1), jnp.float32),
    )(per_row.reshape(128, 128))
    return out[0, 0]


# speedup vs baseline: 1.7251x; 1.7251x over previous
import functools

import jax
import jax.numpy as jnp
from jax import lax
from jax.experimental import pallas as pl
from jax.experimental.pallas import tpu as pltpu
from jax.experimental.pallas import tpu_sc as plsc

_S = 30.0
_LOG2E = 1.4426950408889634
_LN2 = 0.6931471805599453
_K = _S * _LOG2E
_B = 16384
_C = 100
_L = 16
_NC = 2
_NS = 16
_NW = _NC * _NS
_RW = _B // _NW
_G = _RW // _L


def _ln(s):
    bits = plsc.bitcast(s, jnp.int32)
    e = (bits >> 23) - 127
    mant = plsc.bitcast((bits & 0x7FFFFF) | 0x3F800000, jnp.float32)
    u = mant - 1.0
    y = e.astype(jnp.float32) * 0.6931472 + u * (1.0 - u * (0.5 - u * 0.33333334))
    for _ in range(3):
        y = y + s * jnp.exp(-y) - 1.0
    return y


def _sc_body(x_hbm, m_hbm, t_hbm, out_hbm, x_v, t_v, m_v, mx_v, o_v):
    wid = lax.axis_index("s") * _NC + lax.axis_index("c")
    base = wid * _RW
    pltpu.sync_copy(x_hbm.at[pl.ds(base, _RW), :], x_v)
    pltpu.sync_copy(t_hbm.at[pl.ds(base, _RW)], t_v)
    pltpu.sync_copy(m_hbm, m_v)
    lanes = lax.iota(jnp.int32, _L)

    def diag(rows, d):
        col_c = (lanes + d) % _C
        return plsc.load_gather(x_v, [rows, col_c])

    def pass1(g, carry):
        r0 = g * _L
        rows = lanes + r0

        m0 = m1 = m2 = m3 = jnp.full((_L,), -3.0e38, jnp.float32)
        for c in range(0, _C, 4):
            a = diag(rows, c)
            b = diag(rows, c + 1)
            d = diag(rows, c + 2)
            e = diag(rows, c + 3)
            m0 = jnp.maximum(m0, a)
            m1 = jnp.maximum(m1, b)
            m2 = jnp.maximum(m2, d)
            m3 = jnp.maximum(m3, e)
        mx = jnp.maximum(jnp.maximum(m0, m1), jnp.maximum(m2, m3))
        mx_v[pl.ds(r0, _L)] = mx
        return carry

    lax.fori_loop(0, _G, pass1, 0)

    def pass2(g, carry):
        r0 = g * _L
        rows = lanes + r0
        t = t_v[pl.ds(r0, _L)]
        bm = plsc.load_gather(m_v, [t])
        mx = mx_v[pl.ds(r0, _L)]

        s0 = s1 = s2 = s3 = jnp.zeros((_L,), jnp.float32)
        for c in range(0, _C, 4):
            a = diag(rows, c)
            b = diag(rows, c + 1)
            d = diag(rows, c + 2)
            e = diag(rows, c + 3)
            s0 = s0 + jnp.exp((a - mx) * _S)
            s1 = s1 + jnp.exp((b - mx) * _S)
            s2 = s2 + jnp.exp((d - mx) * _S)
            s3 = s3 + jnp.exp((e - mx) * _S)
        s = (s0 + s1) + (s2 + s3)

        xt = plsc.load_gather(x_v, [rows, t])
        e_raw = jnp.exp((xt - mx) * _S)
        e_mod = jnp.exp((xt - bm - mx) * _S)
        s = jnp.maximum(s - e_raw + e_mod, 1e-30)

        loss = _ln(s) + _S * ((mx - xt) + bm)
        o_v[pl.ds(r0, _L)] = loss
        return carry

    lax.fori_loop(0, _G, pass2, 0)
    pltpu.sync_copy(o_v, out_hbm.at[pl.ds(base, _RW)])


_sc_ldam = functools.partial(
    pl.kernel,
    out_type=jax.ShapeDtypeStruct((_B,), jnp.float32),
    mesh=plsc.VectorSubcoreMesh(
        core_axis_name="c", subcore_axis_name="s", num_cores=_NC, num_subcores=_NS
    ),
    scratch_types=[
        pltpu.VMEM((_RW, _C), jnp.float32),
        pltpu.VMEM((_RW,), jnp.int32),
        pltpu.VMEM((_C,), jnp.float32),
        pltpu.VMEM((_RW,), jnp.float32),
        pltpu.VMEM((_RW,), jnp.float32),
    ],
    compiler_params=pltpu.CompilerParams(needs_layout_passes=False),
)(_sc_body)


def _mean_body(x_ref, out_ref):
    out_ref[0, 0] = jnp.sum(x_ref[...]) * (1.0 / _B)


def kernel(inputs, m_list, targets):
    per_row = _sc_ldam(inputs, m_list, targets)
    out = pl.pallas_call(
        _mean_body,
        out_specs=pl.BlockSpec(memory_space=pltpu.SMEM),
        out_shape=jax.ShapeDtypeStruct((1, 1), jnp.float32),
    )(per_row.reshape(128, 128))
    return out[0, 0]
